# baseline (speedup 1.0000x reference)
import functools

import jax
import jax.numpy as jnp
from jax import lax
from jax.experimental import pallas as pl
from jax.experimental.pallas import tpu as pltpu
from jax.experimental.pallas import tpu_sc as plsc

B = 16384
D = 128
NC, NS = 2, 16
NW = NC * NS
CHUNK = 128
STAGES = 1
NBUF = 6


def _gather3(u_idx, i_idx, t_idx, user_table, item_table, time_table, n):
    mesh = plsc.VectorSubcoreMesh(core_axis_name="c", subcore_axis_name="s")
    out_type = [jax.ShapeDtypeStruct((n, D), jnp.float32)] * 3
    cpw = n // (NW * CHUNK)
    npw = cpw * CHUNK
    nch = 3 * cpw
    ahead = 2

    @functools.partial(
        pl.kernel,
        mesh=mesh,
        out_type=out_type,
        scratch_types=[
            pltpu.VMEM((npw,), jnp.int32),
            pltpu.VMEM((npw,), jnp.int32),
            pltpu.VMEM((npw,), jnp.int32),
            pltpu.VMEM((NBUF, CHUNK, D), jnp.float32),
            pltpu.SemaphoreType.DMA,
            pltpu.SemaphoreType.DMA,
            pltpu.SemaphoreType.DMA,
        ],
    )
    def gather_kernel(u_idx, i_idx, t_idx, utab, itab, ttab,
                      uo, io, to, idx_u, idx_i, idx_t, rows,
                      sem_i, sem_g, sem_w):
        wid = lax.axis_index("s") * NC + lax.axis_index("c")
        idxs = (u_idx, i_idx, t_idx)
        idxv = (idx_u, idx_i, idx_t)
        tabs = (utab, itab, ttab)
        outs = (uo, io, to)
        cps = [pltpu.async_copy(idxs[t].at[pl.ds(wid * npw, npw)],
                                idxv[t], sem_i) for t in range(3)]
        for cp in cps:
            cp.wait()

        def tj(c):
            return c % 3, c // 3

        gat = [None] * nch
        wrt = [None] * nch

        def start_gather(c):
            t, j = tj(c)
            gat[c] = pltpu.async_copy(
                tabs[t].at[idxv[t].at[pl.ds(j * CHUNK, CHUNK)]],
                rows.at[c % NBUF], sem_g)

        for c in range(min(ahead, nch)):
            start_gather(c)
        for c in range(nch):
            gat[c].wait()
            t, j = tj(c)
            row = wid * cpw + j
            wrt[c] = pltpu.async_copy(rows.at[c % NBUF],
                                      outs[t].at[pl.ds(row * CHUNK, CHUNK)],
                                      sem_w)
            nxt = c + ahead
            if nxt < nch:
                if nxt >= NBUF:
                    wrt[nxt - NBUF].wait()
                start_gather(nxt)
        for c in range(max(0, nch - NBUF), nch):
            wrt[c].wait()

    return gather_kernel(u_idx, i_idx, t_idx, user_table, item_table,
                         time_table)


BB = 2048


def _dense_body(u_ref, i_ref, t_ref, Wu_ref, bu_ref, Wi_ref, bi_ref,
                Wt_ref, bt_ref, o_ref):
    dn = (((1,), (1,)), ((), ()))
    a = lax.dot_general(Wu_ref[...], u_ref[...], dn,
                        preferred_element_type=jnp.float32) + bu_ref[...][:, None]
    b = lax.dot_general(Wi_ref[...], i_ref[...], dn,
                        preferred_element_type=jnp.float32) + bi_ref[...][:, None]
    c = lax.dot_general(Wt_ref[...], t_ref[...], dn,
                        preferred_element_type=jnp.float32) + bt_ref[...][:, None]
    o_ref[...] = jax.nn.sigmoid(jnp.sum(a * b * c, axis=0))


def _dense(u_rows, i_rows, t_rows, Wu, bu, Wi, bi, Wt, bt, n):
    grid = (n // BB,)
    row_spec = pl.BlockSpec((BB, D), lambda i: (i, 0))
    w_spec = pl.BlockSpec((D, D), lambda i: (0, 0))
    b_spec = pl.BlockSpec((D,), lambda i: (0,))
    return pl.pallas_call(
        _dense_body,
        grid=grid,
        in_specs=[row_spec, row_spec, row_spec,
                  w_spec, b_spec, w_spec, b_spec, w_spec, b_spec],
        out_specs=pl.BlockSpec((BB,), lambda i: (i,)),
        out_shape=jax.ShapeDtypeStruct((n,), jnp.float32),
    )(u_rows, i_rows, t_rows, Wu, bu, Wi, bi, Wt, bt)


def kernel(user, item, time, user_table, item_table, time_table,
           Wu, bu, Wi, bi, Wt, bt):
    u_idx = user.astype(jnp.int32)
    i_idx = item.astype(jnp.int32)
    t_idx = time.astype(jnp.int32)
    n = B // STAGES
    gathered = []
    for s in range(STAGES):
        sl = slice(s * n, (s + 1) * n)
        gathered.append(_gather3(u_idx[sl], i_idx[sl], t_idx[sl],
                                 user_table, item_table, time_table, n))
    preds = [_dense(*g, Wu, bu, Wi, bi, Wt, bt, n) for g in gathered]
    if STAGES == 1:
        return preds[0]
    return jnp.concatenate(preds)


# speedup vs baseline: 1.0589x; 1.0589x over previous
import functools

import jax
import jax.numpy as jnp
from jax import lax
from jax.experimental import pallas as pl
from jax.experimental.pallas import tpu as pltpu
from jax.experimental.pallas import tpu_sc as plsc

B = 16384
D = 128
NC, NS = 2, 16
NW = NC * NS
CHUNK = 128
STAGES = 1
NBUF = 6


def _gather3(u_idx, i_idx, t_idx, user_table, item_table, time_table, n):
    mesh = plsc.VectorSubcoreMesh(core_axis_name="c", subcore_axis_name="s")
    out_type = [jax.ShapeDtypeStruct((n, D), jnp.float32)] * 3
    cpw = n // (NW * CHUNK)
    npw = cpw * CHUNK
    nch = 3 * cpw
    ahead = 2

    @functools.partial(
        pl.kernel,
        mesh=mesh,
        out_type=out_type,
        scratch_types=[
            pltpu.VMEM((npw,), jnp.int32),
            pltpu.VMEM((npw,), jnp.int32),
            pltpu.VMEM((npw,), jnp.int32),
            pltpu.VMEM((NBUF, CHUNK, D), jnp.float32),
            pltpu.SemaphoreType.DMA,
            pltpu.SemaphoreType.DMA,
            pltpu.SemaphoreType.DMA,
        ],
    )
    def gather_kernel(u_idx, i_idx, t_idx, utab, itab, ttab,
                      uo, io, to, idx_u, idx_i, idx_t, rows,
                      sem_i, sem_g, sem_w):
        wid = lax.axis_index("s") * NC + lax.axis_index("c")
        idxs = (u_idx, i_idx, t_idx)
        idxv = (idx_u, idx_i, idx_t)
        tabs = (utab, itab, ttab)
        outs = (uo, io, to)
        cps = [pltpu.async_copy(idxs[t].at[pl.ds(wid * npw, npw)],
                                idxv[t], sem_i) for t in range(3)]
        for cp in cps:
            cp.wait()

        def tj(c):
            return c % 3, c // 3

        gat = [None] * nch
        wrt = [None] * nch

        def start_gather(c):
            t, j = tj(c)
            gat[c] = pltpu.async_copy(
                tabs[t].at[idxv[t].at[pl.ds(j * CHUNK, CHUNK)]],
                rows.at[c % NBUF], sem_g)

        for c in range(min(ahead, nch)):
            start_gather(c)
        for c in range(nch):
            gat[c].wait()
            t, j = tj(c)
            row = wid * cpw + j
            wrt[c] = pltpu.async_copy(rows.at[c % NBUF],
                                      outs[t].at[pl.ds(row * CHUNK, CHUNK)],
                                      sem_w)
            nxt = c + ahead
            if nxt < nch:
                if nxt >= NBUF:
                    wrt[nxt - NBUF].wait()
                start_gather(nxt)
        for c in range(max(0, nch - NBUF), nch):
            wrt[c].wait()

    return gather_kernel(u_idx, i_idx, t_idx, user_table, item_table,
                         time_table)


BB = 4096


def _dense_body(u_ref, i_ref, t_ref, Wu_ref, bu_ref, Wi_ref, bi_ref,
                Wt_ref, bt_ref, o_ref):
    dn = (((1,), (1,)), ((), ()))
    a = lax.dot_general(Wu_ref[...], u_ref[...], dn,
                        preferred_element_type=jnp.float32) + bu_ref[...][:, None]
    b = lax.dot_general(Wi_ref[...], i_ref[...], dn,
                        preferred_element_type=jnp.float32) + bi_ref[...][:, None]
    c = lax.dot_general(Wt_ref[...], t_ref[...], dn,
                        preferred_element_type=jnp.float32) + bt_ref[...][:, None]
    o_ref[...] = jax.nn.sigmoid(jnp.sum(a * b * c, axis=0))


def _dense(u_rows, i_rows, t_rows, Wu, bu, Wi, bi, Wt, bt, n):
    grid = (n // BB,)
    row_spec = pl.BlockSpec((BB, D), lambda i: (i, 0))
    w_spec = pl.BlockSpec((D, D), lambda i: (0, 0))
    b_spec = pl.BlockSpec((D,), lambda i: (0,))
    return pl.pallas_call(
        _dense_body,
        grid=grid,
        in_specs=[row_spec, row_spec, row_spec,
                  w_spec, b_spec, w_spec, b_spec, w_spec, b_spec],
        out_specs=pl.BlockSpec((BB,), lambda i: (i,)),
        out_shape=jax.ShapeDtypeStruct((n,), jnp.float32),
    )(u_rows, i_rows, t_rows, Wu, bu, Wi, bi, Wt, bt)


def kernel(user, item, time, user_table, item_table, time_table,
           Wu, bu, Wi, bi, Wt, bt):
    u_idx = user.astype(jnp.int32)
    i_idx = item.astype(jnp.int32)
    t_idx = time.astype(jnp.int32)
    n = B // STAGES
    gathered = []
    for s in range(STAGES):
        sl = slice(s * n, (s + 1) * n)
        gathered.append(_gather3(u_idx[sl], i_idx[sl], t_idx[sl],
                                 user_table, item_table, time_table, n))
    preds = [_dense(*g, Wu, bu, Wi, bi, Wt, bt, n) for g in gathered]
    if STAGES == 1:
        return preds[0]
    return jnp.concatenate(preds)
